# baseline (speedup 1.0000x reference)
import functools

import jax
import jax.numpy as jnp
from jax.experimental import pallas as pl
from jax.experimental.pallas import tpu as pltpu

_NB, _NP = 10000, 5000
_HID, _H1, _H2 = 128, 2, 2
_STRIDES = (2, 1, 2, 1, 2, 1)


def _ffn(layers, x):
    (W1, b1), (W2, b2), (W3, b3) = layers
    x = jax.nn.relu(x @ W1 + b1)
    x = jax.nn.relu(x @ W2 + b2)
    return x @ W3 + b3


def _cnn(p, x):
    for (W, b), s in zip(p['convs'], _STRIDES):
        x = jax.lax.conv_general_dilated(x, W, (s, s, s), 'SAME',
                                         dimension_numbers=('NCDHW', 'OIDHW', 'NCDHW'))
        x = jax.nn.relu(x + b[None, :, None, None, None])
    x = x.reshape(x.shape[0], -1)
    W, b = p['fc']
    return x @ W + b


def _gatv2(p, x_src, x_dst, ei, heads, out_ch):
    src, dst = ei[0], ei[1]
    n_dst = x_dst.shape[0]
    xl = (x_src @ p['Wl']).reshape(-1, heads, out_ch)
    xr = (x_dst @ p['Wr']).reshape(-1, heads, out_ch)
    m = jax.nn.leaky_relu(xl[src] + xr[dst], 0.2)
    e = jnp.sum(m * p['att'][None], axis=-1)
    emax = jax.ops.segment_max(jax.lax.stop_gradient(e), dst, num_segments=n_dst)
    emax = jnp.where(jnp.isfinite(emax), emax, 0.0)
    ex = jnp.exp(e - emax[dst])
    denom = jax.ops.segment_sum(ex, dst, num_segments=n_dst)
    alpha = ex / (denom[dst] + 1e-16)
    out = jax.ops.segment_sum(alpha[:, :, None] * xl[src], dst, num_segments=n_dst)
    return out.reshape(n_dst, heads * out_ch) + p['bias']


def _hetero(pg, xb, xp, e_bb, e_pp, e_bp, e_pb, heads, out_ch):
    ob = _gatv2(pg['bb'], xb, xb, e_bb, heads, out_ch) + _gatv2(pg['pb'], xp, xb, e_pb, heads, out_ch)
    op = _gatv2(pg['pp'], xp, xp, e_pp, heads, out_ch) + _gatv2(pg['bp'], xb, xp, e_bp, heads, out_ch)
    return ob, op


def _proj_kernel(x_ref, w_ref, b_ref, o_ref):
    o_ref[...] = x_ref[...] @ w_ref[...] + b_ref[...]


def _final_proj(xp, W, b):
    n, d = xp.shape
    return pl.pallas_call(
        _proj_kernel,
        out_shape=jax.ShapeDtypeStruct((n, W.shape[1]), jnp.float32),
    )(xp, W, b[None, :])


def kernel(x_branch, x_proposal, img, params, edge_index_bb, edge_index_pp,
           edge_index_bp, edge_index_pb):
    xb = _ffn(params['ffn_branch'], x_branch)
    xp = jnp.concatenate([_ffn(params['ffn_prop'], x_proposal), _cnn(params['cnn'], img)], axis=1)
    xb, xp = _hetero(params['gat1'], xb, xp, edge_index_bb, edge_index_pp,
                     edge_index_bp, edge_index_pb, _H1, _HID)
    xb, xp = _hetero(params['gat2'], xb, xp, edge_index_bb, edge_index_pp,
                     edge_index_bp, edge_index_pb, _H2, _HID * _H1)
    W, b = params['out']
    return _final_proj(xp, W, b)


# speedup vs baseline: 7.6920x; 7.6920x over previous
import functools

import jax
import jax.numpy as jnp
from jax import lax
from jax.experimental import pallas as pl
from jax.experimental.pallas import tpu as pltpu
from jax.experimental.pallas import tpu_sc as plsc

_NB, _NP = 10000, 5000
_HID, _H1, _H2 = 128, 2, 2
_STRIDES = (2, 1, 2, 1, 2, 1)

_NTILE = 16
_BLK = 128
_SEG = 128


def _rup(x, m):
    return (x + m - 1) // m * m


_GDN = lax.GatherDimensionNumbers(offset_dims=(), collapsed_slice_dims=(0,),
                                  start_index_map=(0,))


def _perm16(v, idx):
    return lax.gather(v, idx[:, None], _GDN, (1,),
                      mode=lax.GatherScatterMode.PROMISE_IN_BOUNDS)


def _hsum16(v):
    lane = lax.broadcasted_iota(jnp.int32, (16,), 0)
    for k in (8, 4, 2, 1):
        v = v + _perm16(v, lane ^ k)
    return v


@functools.cache
def _gat_sc_kernel(n_src, n_dst, e_pad, c):
    n_dst_pad = _rup(n_dst, 128)
    nseg = c // _SEG
    ept = e_pad // _NTILE
    nblk = ept // _BLK
    assert ept % _BLK == 0
    nch = _SEG // 16
    rpt = n_dst_pad // _NTILE

    mesh = plsc.VectorSubcoreMesh(core_axis_name="c", subcore_axis_name="s")
    out_types = [jax.ShapeDtypeStruct((2, n_dst_pad, _SEG), jnp.float32)
                 for _ in range(nseg)]
    out_types.append(jax.ShapeDtypeStruct((2, e_pad), jnp.float32))

    scratch = dict(
        src_i=pltpu.VMEM((_BLK,), jnp.int32),
        dst_i=pltpu.VMEM((_BLK,), jnp.int32),
        gr_i=pltpu.VMEM((_BLK,), jnp.int32),
        p=pltpu.VMEM((_BLK, _SEG), jnp.float32),
        q=pltpu.VMEM((_BLK, _SEG), jnp.float32),
        attv=pltpu.VMEM((c + 16,), jnp.float32),
        enosh=pltpu.VMEM((_BLK,), jnp.float32),
        exblk=pltpu.VMEM((_BLK,), jnp.float32),
        accum=pltpu.VMEM_SHARED((n_dst_pad, _SEG), jnp.float32),
        sem0=pltpu.SemaphoreType.DMA,
        sem1=pltpu.SemaphoreType.DMA,
    )

    def body(xlt, xrt, att, src, dst, *outs, **s):
        ex_out = outs[-1]
        lane = lax.broadcasted_iota(jnp.int32, (16,), 0)
        h = lax.axis_index("c")
        t = lax.axis_index("s")

        pltpu.sync_copy(att.at[h], s["attv"])
        attc = [s["attv"][pl.ds(cc * 16, 16)] for cc in range(c // 16 + 1)]

        def zero_accum():
            def zp(r, _):
                for cc in range(nch):
                    s["p"][r, pl.ds(cc * 16, 16)] = jnp.zeros((16,), jnp.float32)
                return 0
            lax.fori_loop(0, _BLK, zp, 0)
            base = t * rpt
            done = 0
            while done < rpt:
                step = min(_BLK, rpt - done)
                pltpu.sync_copy(s["p"].at[pl.ds(0, step)],
                                s["accum"].at[pl.ds(base + done, step)])
                done += step

        def flush(dst_hbm):
            base = t * rpt
            done = 0
            while done < rpt:
                step = min(512, rpt - done)
                pltpu.sync_copy(s["accum"].at[pl.ds(base + done, step)],
                                dst_hbm.at[h, pl.ds(base + done, step)])
                done += step

        def sweep(seg, kind, add_partial):
            def block_body(b, _):
                ebase = t * ept + b * _BLK
                pltpu.sync_copy(src.at[pl.ds(ebase, _BLK)], s["src_i"])
                pltpu.sync_copy(dst.at[pl.ds(ebase, _BLK)], s["dst_i"])

                def gidx(g, _):
                    s16 = s["src_i"][pl.ds(g * 16, 16)]
                    d16 = s["dst_i"][pl.ds(g * 16, 16)]
                    s["src_i"][pl.ds(g * 16, 16)] = \
                        s16 * (2 * nseg) + h * nseg + seg
                    s["gr_i"][pl.ds(g * 16, 16)] = \
                        d16 * (2 * (nseg + 1)) + h * (nseg + 1) + seg
                    return 0
                lax.fori_loop(0, 8, gidx, 0)

                cp = pltpu.async_copy(xlt.at[s["src_i"]], s["p"], s["sem0"])
                if kind != 2:
                    pltpu.async_copy(xrt.at[s["gr_i"]], s["q"], s["sem1"]).wait()
                if kind != 0 and (add_partial or kind == 2):
                    pltpu.sync_copy(ex_out.at[h, pl.ds(ebase, _BLK)],
                                    s["enosh"] if kind == 1 else s["exblk"])
                cp.wait()

                ac = [attc[seg * nch + cc] for cc in range(nch)]

                if kind != 2:
                    def logit(g, _):
                        exg = jnp.zeros((16,), jnp.float32)
                        for j in range(16):
                            row = g * 16 + j
                            acc = None
                            for cc in range(nch):
                                z = (s["p"][row, pl.ds(cc * 16, 16)] +
                                     s["q"][row, pl.ds(cc * 16, 16)])
                                m = jnp.maximum(z, 0.2 * z)
                                term = m * ac[cc]
                                acc = term if acc is None else acc + term
                            e_spl = _hsum16(acc)
                            exg = jnp.where(lane == j, e_spl, exg)
                        if kind == 1 and add_partial:
                            exg = exg + s["enosh"][pl.ds(g * 16, 16)]
                        s["enosh"][pl.ds(g * 16, 16)] = exg
                        return 0
                    lax.fori_loop(0, 8, logit, 0)

                if kind == 0:
                    pltpu.sync_copy(s["enosh"], ex_out.at[h, pl.ds(ebase, _BLK)])
                    return 0

                if kind == 1:
                    def gsh(g, _):
                        s["gr_i"][pl.ds(g * 16, 16)] = \
                            s["gr_i"][pl.ds(g * 16, 16)] + (nseg - seg)
                        return 0
                    lax.fori_loop(0, 8, gsh, 0)
                    pltpu.async_copy(xrt.at[s["gr_i"]], s["q"], s["sem1"]).wait()

                    def finish(g, _):
                        eg = s["enosh"][pl.ds(g * 16, 16)]
                        exg = jnp.zeros((16,), jnp.float32)
                        for j in range(16):
                            row = g * 16 + j
                            zs = s["q"][row, pl.ds(0, 16)]
                            ms = jnp.maximum(zs, 0.2 * zs)
                            sh = _hsum16(ms * attc[c // 16])
                            e_spl = _perm16(eg, jnp.full((16,), j, jnp.int32))
                            exb = jnp.exp(e_spl + sh)
                            for cc in range(nch):
                                s["p"][row, pl.ds(cc * 16, 16)] = \
                                    exb * s["p"][row, pl.ds(cc * 16, 16)]
                            exg = jnp.where(lane == j, exb, exg)
                        s["exblk"][pl.ds(g * 16, 16)] = exg
                        return 0
                    lax.fori_loop(0, 8, finish, 0)
                else:
                    def replay(g, _):
                        exg16 = s["exblk"][pl.ds(g * 16, 16)]
                        for j in range(16):
                            row = g * 16 + j
                            exb = _perm16(exg16, jnp.full((16,), j, jnp.int32))
                            for cc in range(nch):
                                s["p"][row, pl.ds(cc * 16, 16)] = \
                                    exb * s["p"][row, pl.ds(cc * 16, 16)]
                        return 0
                    lax.fori_loop(0, 8, replay, 0)

                pltpu.sync_copy(s["p"], s["accum"].at[s["dst_i"]], add=True)
                if kind == 1:
                    pltpu.sync_copy(s["exblk"], ex_out.at[h, pl.ds(ebase, _BLK)])
                return 0
            lax.fori_loop(0, nblk, block_body, 0)

        if nseg == 1:
            zero_accum()
            plsc.subcore_barrier()
            sweep(0, 1, False)
            plsc.subcore_barrier()
            flush(outs[0])
        else:
            zero_accum()
            plsc.subcore_barrier()
            sweep(0, 0, False)
            sweep(1, 1, True)
            plsc.subcore_barrier()
            flush(outs[0])
            plsc.subcore_barrier()
            zero_accum()
            plsc.subcore_barrier()
            sweep(0, 2, False)
            plsc.subcore_barrier()
            flush(outs[1])

    return pl.kernel(body, out_type=out_types, mesh=mesh,
                     scratch_types=scratch)


def _gatv2_sc(p, x_src, x_dst, ei, heads, out_ch):
    assert heads == 2
    n_src, n_dst = x_src.shape[0], x_dst.shape[0]
    c = out_ch
    nseg = c // _SEG
    n_dst_pad = _rup(n_dst, 128)
    src, dst = ei[0], ei[1]
    e = src.shape[0]
    e_pad = _rup(e, _NTILE * _BLK)

    xl = x_src @ p['Wl']
    xr = x_dst @ p['Wr']
    att = p['att']

    att_p = jnp.maximum(att, 0.0)
    att_n = jnp.maximum(-att, 0.0)
    xl3 = xl.reshape(n_src, 2, c)
    xr3 = xr.reshape(n_dst, 2, c)
    u = jnp.einsum('nhc,hc->nh', jax.nn.relu(xl3), att_p) + \
        0.2 * jnp.einsum('nhc,hc->nh', jax.nn.relu(-xl3), att_n)
    v = jnp.einsum('nhc,hc->nh', jax.nn.relu(xr3), att_p) + \
        0.2 * jnp.einsum('nhc,hc->nh', jax.nn.relu(-xr3), att_n)
    shift = v + jnp.max(u, axis=0, keepdims=True)

    xlt = xl3.reshape(n_src * 2 * nseg, _SEG)
    shift_pad = jnp.full((n_dst_pad + 8, 2), 1e30, jnp.float32)
    shift_pad = shift_pad.at[:n_dst].set(shift)
    xr_pad = jnp.pad(xr3, ((0, n_dst_pad + 8 - n_dst), (0, 0), (0, 0)))
    xr_seg = xr_pad.reshape(n_dst_pad + 8, 2, nseg, _SEG)
    sh_seg = jnp.concatenate(
        [shift_pad[:, :, None, None],
         jnp.zeros((n_dst_pad + 8, 2, 1, _SEG - 1), jnp.float32)], axis=3)
    xrt = jnp.concatenate([xr_seg, sh_seg], axis=2)
    xrt = xrt.reshape((n_dst_pad + 8) * 2 * (nseg + 1), _SEG)

    att_x = jnp.concatenate(
        [att, jnp.full((2, 1), -1.0, jnp.float32),
         jnp.zeros((2, 15), jnp.float32)], axis=1)

    src_p = jnp.pad(src, (0, e_pad - e))
    dst_p = jnp.pad(dst, (0, e_pad - e), constant_values=n_dst)

    kern = _gat_sc_kernel(n_src, n_dst, e_pad, c)
    outs = kern(xlt, xrt, att_x, src_p, dst_p)
    ex = outs[-1][:, :e]
    den = jax.ops.segment_sum(ex.T, dst, num_segments=n_dst)
    den = den.T[:, :, None] + 1e-30
    if nseg == 1:
        num = outs[0]
    else:
        num = jnp.concatenate([outs[1], outs[0]], axis=-1)
    res = num[:, :n_dst, :] / den
    return res.transpose(1, 0, 2).reshape(n_dst, 2 * c) + p['bias']


def _ffn(layers, x):
    (w1, b1), (w2, b2), (w3, b3) = layers
    x = jax.nn.relu(x @ w1 + b1)
    x = jax.nn.relu(x @ w2 + b2)
    return x @ w3 + b3


def _cnn(p, x):
    for (w, b), s in zip(p['convs'], _STRIDES):
        x = jax.lax.conv_general_dilated(x, w, (s, s, s), 'SAME',
                                         dimension_numbers=('NCDHW', 'OIDHW', 'NCDHW'))
        x = jax.nn.relu(x + b[None, :, None, None, None])
    x = x.reshape(x.shape[0], -1)
    w, b = p['fc']
    return x @ w + b


def _hetero(pg, xb, xp, e_bb, e_pp, e_bp, e_pb, heads, out_ch):
    ob = _gatv2_sc(pg['bb'], xb, xb, e_bb, heads, out_ch) + \
        _gatv2_sc(pg['pb'], xp, xb, e_pb, heads, out_ch)
    op = _gatv2_sc(pg['pp'], xp, xp, e_pp, heads, out_ch) + \
        _gatv2_sc(pg['bp'], xb, xp, e_bp, heads, out_ch)
    return ob, op


def kernel(x_branch, x_proposal, img, params, edge_index_bb, edge_index_pp,
           edge_index_bp, edge_index_pb):
    xb = _ffn(params['ffn_branch'], x_branch)
    xp = jnp.concatenate([_ffn(params['ffn_prop'], x_proposal),
                          _cnn(params['cnn'], img)], axis=1)
    xb, xp = _hetero(params['gat1'], xb, xp, edge_index_bb, edge_index_pp,
                     edge_index_bp, edge_index_pb, _H1, _HID)
    xb, xp = _hetero(params['gat2'], xb, xp, edge_index_bb, edge_index_pp,
                     edge_index_bp, edge_index_pb, _H2, _HID * _H1)
    w, b = params['out']
    return xp @ w + b


# speedup vs baseline: 7.8428x; 1.0196x over previous
import functools

import jax
import jax.numpy as jnp
from jax import lax
from jax.experimental import pallas as pl
from jax.experimental.pallas import tpu as pltpu
from jax.experimental.pallas import tpu_sc as plsc

_NB, _NP = 10000, 5000
_HID, _H1, _H2 = 128, 2, 2
_STRIDES = (2, 1, 2, 1, 2, 1)

_NTILE = 16
_BLK = 128
_SEG = 128


def _rup(x, m):
    return (x + m - 1) // m * m


_GDN = lax.GatherDimensionNumbers(offset_dims=(), collapsed_slice_dims=(0,),
                                  start_index_map=(0,))


def _perm16(v, idx):
    return lax.gather(v, idx[:, None], _GDN, (1,),
                      mode=lax.GatherScatterMode.PROMISE_IN_BOUNDS)


def _hsum16(v):
    lane = lax.broadcasted_iota(jnp.int32, (16,), 0)
    for k in (8, 4, 2, 1):
        v = v + _perm16(v, lane ^ k)
    return v


@functools.cache
def _gat_sc_kernel(n_src, n_dst, e_pad, c):
    n_dst_pad = _rup(n_dst, 128)
    nseg = c // _SEG
    ept = e_pad // _NTILE
    nblk = ept // _BLK
    assert ept % _BLK == 0
    nch = _SEG // 16
    rpt = n_dst_pad // _NTILE

    mesh = plsc.VectorSubcoreMesh(core_axis_name="c", subcore_axis_name="s")
    out_types = [jax.ShapeDtypeStruct((2, n_dst_pad, _SEG), jnp.float32)
                 for _ in range(nseg)]
    out_types.append(jax.ShapeDtypeStruct((2, e_pad), jnp.float32))

    scratch = dict(
        src_i=pltpu.VMEM((_BLK,), jnp.int32),
        dst_i=pltpu.VMEM((_BLK,), jnp.int32),
        gr_i=pltpu.VMEM((_BLK,), jnp.int32),
        p=pltpu.VMEM((_BLK, _SEG), jnp.float32),
        q=pltpu.VMEM((_BLK, _SEG), jnp.float32),
        attv=pltpu.VMEM((c,), jnp.float32),
        enosh=pltpu.VMEM((_BLK,), jnp.float32),
        exblk=pltpu.VMEM((_BLK,), jnp.float32),
        accum=pltpu.VMEM_SHARED((n_dst_pad, _SEG), jnp.float32),
        sem0=pltpu.SemaphoreType.DMA,
        sem1=pltpu.SemaphoreType.DMA,
    )

    def body(xlt, xrt, att, src, dst, *outs, **s):
        ex_out = outs[-1]
        lane = lax.broadcasted_iota(jnp.int32, (16,), 0)
        h = lax.axis_index("c")
        t = lax.axis_index("s")

        pltpu.sync_copy(att.at[h], s["attv"])
        attc = [s["attv"][pl.ds(cc * 16, 16)] for cc in range(c // 16)]

        def zero_accum():
            def zp(r, _):
                for cc in range(nch):
                    s["p"][r, pl.ds(cc * 16, 16)] = jnp.zeros((16,), jnp.float32)
                return 0
            lax.fori_loop(0, _BLK, zp, 0)
            base = t * rpt
            done = 0
            while done < rpt:
                step = min(_BLK, rpt - done)
                pltpu.sync_copy(s["p"].at[pl.ds(0, step)],
                                s["accum"].at[pl.ds(base + done, step)])
                done += step

        def flush(dst_hbm):
            base = t * rpt
            done = 0
            while done < rpt:
                step = min(512, rpt - done)
                pltpu.sync_copy(s["accum"].at[pl.ds(base + done, step)],
                                dst_hbm.at[h, pl.ds(base + done, step)])
                done += step

        def sweep(seg, kind, add_partial):
            def block_body(b, _):
                ebase = t * ept + b * _BLK
                pltpu.sync_copy(src.at[pl.ds(ebase, _BLK)], s["src_i"])
                pltpu.sync_copy(dst.at[pl.ds(ebase, _BLK)], s["dst_i"])

                def gidx(g, _):
                    s16 = s["src_i"][pl.ds(g * 16, 16)]
                    d16 = s["dst_i"][pl.ds(g * 16, 16)]
                    s["src_i"][pl.ds(g * 16, 16)] = \
                        s16 * (2 * nseg) + h * nseg + seg
                    s["gr_i"][pl.ds(g * 16, 16)] = \
                        d16 * (2 * (nseg + 1)) + h * (nseg + 1) + seg
                    return 0
                lax.fori_loop(0, 8, gidx, 0)

                cp = pltpu.async_copy(xlt.at[s["src_i"]], s["p"], s["sem0"])
                if kind != 2:
                    pltpu.async_copy(xrt.at[s["gr_i"]], s["q"], s["sem1"]).wait()
                if kind != 0 and (add_partial or kind == 2):
                    pltpu.sync_copy(ex_out.at[h, pl.ds(ebase, _BLK)],
                                    s["enosh"] if kind == 1 else s["exblk"])
                cp.wait()

                ac = [attc[seg * nch + cc] for cc in range(nch)]

                if kind != 2:
                    def logit(g, _):
                        exg = jnp.zeros((16,), jnp.float32)
                        for j in range(16):
                            row = g * 16 + j
                            acc = None
                            for cc in range(nch):
                                z = (s["p"][row, pl.ds(cc * 16, 16)] +
                                     s["q"][row, pl.ds(cc * 16, 16)])
                                m = jnp.maximum(z, 0.2 * z)
                                term = m * ac[cc]
                                acc = term if acc is None else acc + term
                            e_spl = _hsum16(acc)
                            exg = jnp.where(lane == j, e_spl, exg)
                        if kind == 1 and add_partial:
                            exg = exg + s["enosh"][pl.ds(g * 16, 16)]
                        s["enosh"][pl.ds(g * 16, 16)] = exg
                        return 0
                    lax.fori_loop(0, 8, logit, 0)

                if kind == 0:
                    pltpu.sync_copy(s["enosh"], ex_out.at[h, pl.ds(ebase, _BLK)])
                    return 0

                if kind == 1:
                    def gsh(g, _):
                        s["gr_i"][pl.ds(g * 16, 16)] = \
                            s["gr_i"][pl.ds(g * 16, 16)] + (nseg - seg)
                        return 0
                    lax.fori_loop(0, 8, gsh, 0)
                    pltpu.async_copy(xrt.at[s["gr_i"]], s["q"], s["sem1"]).wait()

                    def finish(g, _):
                        eg = s["enosh"][pl.ds(g * 16, 16)]
                        exg = jnp.zeros((16,), jnp.float32)
                        for j in range(16):
                            row = g * 16 + j
                            zs = s["q"][row, pl.ds(0, 16)]
                            e_spl = _perm16(eg, jnp.full((16,), j, jnp.int32))
                            exb = jnp.exp(e_spl - zs)
                            for cc in range(nch):
                                s["p"][row, pl.ds(cc * 16, 16)] = \
                                    exb * s["p"][row, pl.ds(cc * 16, 16)]
                            exg = jnp.where(lane == j, exb, exg)
                        s["exblk"][pl.ds(g * 16, 16)] = exg
                        return 0
                    lax.fori_loop(0, 8, finish, 0)
                else:
                    def replay(g, _):
                        exg16 = s["exblk"][pl.ds(g * 16, 16)]
                        for j in range(16):
                            row = g * 16 + j
                            exb = _perm16(exg16, jnp.full((16,), j, jnp.int32))
                            for cc in range(nch):
                                s["p"][row, pl.ds(cc * 16, 16)] = \
                                    exb * s["p"][row, pl.ds(cc * 16, 16)]
                        return 0
                    lax.fori_loop(0, 8, replay, 0)

                pltpu.sync_copy(s["p"], s["accum"].at[s["dst_i"]], add=True)
                if kind == 1:
                    pltpu.sync_copy(s["exblk"], ex_out.at[h, pl.ds(ebase, _BLK)])
                return 0
            lax.fori_loop(0, nblk, block_body, 0)

        if nseg == 1:
            zero_accum()
            plsc.subcore_barrier()
            sweep(0, 1, False)
            plsc.subcore_barrier()
            flush(outs[0])
        else:
            zero_accum()
            plsc.subcore_barrier()
            sweep(0, 0, False)
            sweep(1, 1, True)
            plsc.subcore_barrier()
            flush(outs[0])
            plsc.subcore_barrier()
            zero_accum()
            plsc.subcore_barrier()
            sweep(0, 2, False)
            plsc.subcore_barrier()
            flush(outs[1])

    return pl.kernel(body, out_type=out_types, mesh=mesh,
                     scratch_types=scratch)


def _gatv2_sc(p, x_src, x_dst, ei, heads, out_ch):
    assert heads == 2
    n_src, n_dst = x_src.shape[0], x_dst.shape[0]
    c = out_ch
    nseg = c // _SEG
    n_dst_pad = _rup(n_dst, 128)
    src, dst = ei[0], ei[1]
    e = src.shape[0]
    e_pad = _rup(e, _NTILE * _BLK)

    xl = x_src @ p['Wl']
    xr = x_dst @ p['Wr']
    att = p['att']

    att_p = jnp.maximum(att, 0.0)
    att_n = jnp.maximum(-att, 0.0)
    xl3 = xl.reshape(n_src, 2, c)
    xr3 = xr.reshape(n_dst, 2, c)
    u = jnp.einsum('nhc,hc->nh', jax.nn.relu(xl3), att_p) + \
        0.2 * jnp.einsum('nhc,hc->nh', jax.nn.relu(-xl3), att_n)
    v = jnp.einsum('nhc,hc->nh', jax.nn.relu(xr3), att_p) + \
        0.2 * jnp.einsum('nhc,hc->nh', jax.nn.relu(-xr3), att_n)
    shift = v + jnp.max(u, axis=0, keepdims=True)

    xlt = xl3.reshape(n_src * 2 * nseg, _SEG)
    shift_pad = jnp.full((n_dst_pad + 8, 2), 1e30, jnp.float32)
    shift_pad = shift_pad.at[:n_dst].set(shift)
    xr_pad = jnp.pad(xr3, ((0, n_dst_pad + 8 - n_dst), (0, 0), (0, 0)))
    xr_seg = xr_pad.reshape(n_dst_pad + 8, 2, nseg, _SEG)
    sh_seg = jnp.concatenate(
        [jnp.broadcast_to(shift_pad[:, :, None, None],
                          (n_dst_pad + 8, 2, 1, 16)),
         jnp.zeros((n_dst_pad + 8, 2, 1, _SEG - 16), jnp.float32)], axis=3)
    xrt = jnp.concatenate([xr_seg, sh_seg], axis=2)
    xrt = xrt.reshape((n_dst_pad + 8) * 2 * (nseg + 1), _SEG)

    src_p = jnp.pad(src, (0, e_pad - e))
    dst_p = jnp.pad(dst, (0, e_pad - e), constant_values=n_dst)

    kern = _gat_sc_kernel(n_src, n_dst, e_pad, c)
    outs = kern(xlt, xrt, att, src_p, dst_p)
    ex = outs[-1][:, :e]
    den = jax.ops.segment_sum(ex.T, dst, num_segments=n_dst)
    den = den.T[:, :, None] + 1e-30
    if nseg == 1:
        num = outs[0]
    else:
        num = jnp.concatenate([outs[1], outs[0]], axis=-1)
    res = num[:, :n_dst, :] / den
    return res.transpose(1, 0, 2).reshape(n_dst, 2 * c) + p['bias']


def _ffn(layers, x):
    (w1, b1), (w2, b2), (w3, b3) = layers
    x = jax.nn.relu(x @ w1 + b1)
    x = jax.nn.relu(x @ w2 + b2)
    return x @ w3 + b3


def _cnn(p, x):
    for (w, b), s in zip(p['convs'], _STRIDES):
        x = jax.lax.conv_general_dilated(x, w, (s, s, s), 'SAME',
                                         dimension_numbers=('NCDHW', 'OIDHW', 'NCDHW'))
        x = jax.nn.relu(x + b[None, :, None, None, None])
    x = x.reshape(x.shape[0], -1)
    w, b = p['fc']
    return x @ w + b


def _hetero(pg, xb, xp, e_bb, e_pp, e_bp, e_pb, heads, out_ch):
    ob = _gatv2_sc(pg['bb'], xb, xb, e_bb, heads, out_ch) + \
        _gatv2_sc(pg['pb'], xp, xb, e_pb, heads, out_ch)
    op = _gatv2_sc(pg['pp'], xp, xp, e_pp, heads, out_ch) + \
        _gatv2_sc(pg['bp'], xb, xp, e_bp, heads, out_ch)
    return ob, op


def kernel(x_branch, x_proposal, img, params, edge_index_bb, edge_index_pp,
           edge_index_bp, edge_index_pb):
    xb = _ffn(params['ffn_branch'], x_branch)
    xp = jnp.concatenate([_ffn(params['ffn_prop'], x_proposal),
                          _cnn(params['cnn'], img)], axis=1)
    xb, xp = _hetero(params['gat1'], xb, xp, edge_index_bb, edge_index_pp,
                     edge_index_bp, edge_index_pb, _H1, _HID)
    xb, xp = _hetero(params['gat2'], xb, xp, edge_index_bb, edge_index_pp,
                     edge_index_bp, edge_index_pb, _H2, _HID * _H1)
    w, b = params['out']
    return xp @ w + b
